# Initial kernel scaffold; baseline (speedup 1.0000x reference)
import jax
import jax.numpy as jnp
from jax.experimental import pallas as pl


def kernel(points_ref, points):
    raise NotImplementedError("write your pallas kernel here")


# speedup vs baseline: 27.5900x; 27.5900x over previous
import functools

import jax
import jax.numpy as jnp
from jax import lax
from jax.experimental import pallas as pl
from jax.experimental.pallas import tpu as pltpu
from jax.experimental.pallas import tpu_sc as plsc

_B, _N, _D = 4, 4096, 3
_NN = 16
_BQ = 256
_DP = 8
_DG = 8
_NW = 32
_WPB = _NW // _B
_QW = _N // _WPB


def _knn_body(q_ref, kT_ref, idx_ref):
    b = pl.program_id(0)
    qi = pl.program_id(1)
    q = q_ref[0]
    kT = kT_ref[0]
    del b, qi
    dot = lax.dot_general(q, kT, (((1,), (0,)), ((), ())),
                          preferred_element_type=jnp.float32)
    sqq = jnp.sum(q * q, axis=1, keepdims=True)
    sqk = jnp.sum(kT * kT, axis=0, keepdims=True)
    d2 = sqq + sqk - 2.0 * dot
    col = lax.broadcasted_iota(jnp.int32, (_BQ, _N), 1)
    i32 = lax.bitcast_convert_type(d2, jnp.int32)
    key = i32 ^ (lax.shift_right_arithmetic(i32, 31) & jnp.int32(0x7FFFFFFF))
    pk = (key & jnp.int32(~0xFFF)) | col
    big = jnp.int32(0x7FFFFFFF)
    outs = []
    for k in range(_NN + 1):
        m = jnp.min(pk, axis=1, keepdims=True)
        if k > 0:
            outs.append(m & jnp.int32(0xFFF))
        pk = jnp.where(pk == m, big, pk)
    idx_ref[0] = jnp.concatenate(outs, axis=1)


def _knn_idx(points_ref):
    ptsp = jnp.pad(points_ref, ((0, 0), (0, 0), (0, _DP - _D)))
    ptsT = jnp.swapaxes(ptsp, 1, 2)
    return pl.pallas_call(
        _knn_body,
        grid=(_B, _N // _BQ),
        in_specs=[
            pl.BlockSpec((1, _BQ, _DP), lambda b, q: (b, q, 0)),
            pl.BlockSpec((1, _DP, _N), lambda b, q: (b, 0, 0)),
        ],
        out_specs=pl.BlockSpec((1, _BQ, _NN), lambda b, q: (b, q, 0)),
        out_shape=jax.ShapeDtypeStruct((_B, _N, _NN), jnp.int32),
    )(ptsp, ptsT)


def _rsqrt(x):
    i = lax.bitcast_convert_type(x, jnp.int32)
    y = lax.bitcast_convert_type(
        jnp.int32(0x5F3759DF) - lax.shift_right_logical(i, jnp.int32(1)),
        jnp.float32)
    for _ in range(3):
        y = y * (1.5 - 0.5 * x * y * y)
    return y


def _sc_loss_body(tpr_hbm, tp_hbm, idx_hbm, out_hbm,
                  pr_v, p_v, idx_v, acc_v):
    wid = lax.axis_index("s") * 2 + lax.axis_index("c")
    b = wid // _WPB
    e = wid % _WPB
    pltpu.sync_copy(tpr_hbm.at[pl.ds(b * _N * _DG, _N * _DG)], pr_v)
    pltpu.sync_copy(tp_hbm.at[pl.ds(b * _N * _DG, _N * _DG)], p_v)
    qbase = b * _N + e * _QW
    pltpu.sync_copy(idx_hbm.at[pl.ds(qbase * _NN, _QW * _NN)], idx_v)

    def row(r, acc):
        ridx = idx_v[pl.ds(r * _NN, _NN)] * _DG
        qoff = (e * _QW + r) * _DG
        s2r = jnp.zeros((16,), jnp.float32)
        s2 = jnp.zeros((16,), jnp.float32)
        for cd in range(_D):
            nbr_off = ridx + jnp.int32(cd)
            q_off = jnp.full((16,), qoff + cd, jnp.int32)
            dr = plsc.load_gather(pr_v, [nbr_off]) - \
                plsc.load_gather(pr_v, [q_off])
            s2r = s2r + dr * dr
            dp = plsc.load_gather(p_v, [nbr_off]) - \
                plsc.load_gather(p_v, [q_off])
            s2 = s2 + dp * dp
        dist_ref = s2r * _rsqrt(jnp.maximum(s2r, 1e-30))
        dist = s2 * _rsqrt(jnp.maximum(s2, 1e-30))
        stretch = jnp.maximum(dist / (dist_ref + 1e-10) - 1.0, 0.0)
        return acc + stretch

    acc = lax.fori_loop(0, _QW, row, jnp.zeros((16,), jnp.float32))
    acc_v[...] = acc
    pltpu.sync_copy(acc_v, out_hbm.at[wid])


def _sc_loss(table_pr, table_p, idx_flat):
    mesh = plsc.VectorSubcoreMesh(core_axis_name="c", subcore_axis_name="s")
    kfn = pl.kernel(
        _sc_loss_body,
        out_type=jax.ShapeDtypeStruct((_NW, 16), jnp.float32),
        mesh=mesh,
        compiler_params=pltpu.CompilerParams(needs_layout_passes=False),
        scratch_types=[
            pltpu.VMEM((_N * _DG,), jnp.float32),
            pltpu.VMEM((_N * _DG,), jnp.float32),
            pltpu.VMEM((_QW * _NN,), jnp.int32),
            pltpu.VMEM((16,), jnp.float32),
        ],
    )
    return kfn(table_pr, table_p, idx_flat)


def kernel(points_ref, points):
    gidx = _knn_idx(points_ref)
    table_pr = jnp.pad(points_ref.reshape(_B * _N, _D),
                       ((0, 0), (0, _DG - _D))).reshape(-1)
    table_p = jnp.pad(points.reshape(_B * _N, _D),
                      ((0, 0), (0, _DG - _D))).reshape(-1)
    partials = _sc_loss(table_pr, table_p, gidx.reshape(-1))
    return jnp.sum(partials) / jnp.float32(_B * _N * _NN)


# speedup vs baseline: 59.6976x; 2.1637x over previous
import functools

import jax
import jax.numpy as jnp
from jax import lax
from jax.experimental import pallas as pl
from jax.experimental.pallas import tpu as pltpu
from jax.experimental.pallas import tpu_sc as plsc

_B, _N, _D = 4, 4096, 3
_NN = 16
_BQ = 512
_Q = 5
_DP = 8
_DG = 8
_NW = 32
_WPB = _NW // _B
_QW = _N // _WPB


def _ce(a, b):
    return jnp.minimum(a, b), jnp.maximum(a, b)


def _merge22(a, b):
    c0, d0 = _ce(a[0], b[0])
    c1, d1 = _ce(a[1], b[1])
    m, mm = _ce(d0, c1)
    return [c0, m, mm, d1]


def _merge44_low5(a, b):
    lo = [jnp.minimum(a[i], b[3 - i]) for i in range(4)]
    hi = [jnp.maximum(a[i], b[3 - i]) for i in range(4)]
    l0, l2 = _ce(lo[0], lo[2])
    l1, l3 = _ce(lo[1], lo[3])
    s0, s1 = _ce(l0, l1)
    s2, s3 = _ce(l2, l3)
    h = jnp.minimum(jnp.minimum(hi[0], hi[1]), jnp.minimum(hi[2], hi[3]))
    return [s0, s1, s2, s3, h]


def _merge55_low5(a, b):
    v = [jnp.minimum(a[i], b[4 - i]) for i in range(5)]
    for i, j in ((0, 1), (3, 4), (2, 4), (2, 3), (1, 4),
                 (0, 3), (0, 2), (1, 3), (1, 2)):
        v[i], v[j] = _ce(v[i], v[j])
    return v


def _top5_tree(tiles):
    p2 = [_ce(tiles[i], tiles[i + 1]) for i in range(0, len(tiles), 2)]
    p4 = [_merge22(p2[i], p2[i + 1]) for i in range(0, len(p2), 2)]
    p5 = [_merge44_low5(p4[i], p4[i + 1]) for i in range(0, len(p4), 2)]
    while len(p5) > 1:
        p5 = [_merge55_low5(p5[i], p5[i + 1]) for i in range(0, len(p5), 2)]
    return p5[0]


def _knn_body(q_ref, kT_ref, idx_ref):
    b = pl.program_id(0)
    qi = pl.program_id(1)
    q = q_ref[0]
    kT = kT_ref[0]
    del b, qi
    dot = lax.dot_general(q, kT, (((1,), (0,)), ((), ())),
                          preferred_element_type=jnp.float32)
    sqq = jnp.sum(q * q, axis=1, keepdims=True)
    sqk = jnp.sum(kT * kT, axis=0, keepdims=True)
    d2 = sqq + sqk - 2.0 * dot
    col = lax.broadcasted_iota(jnp.int32, (_BQ, _N), 1)
    key = lax.bitcast_convert_type(d2 + 0.5, jnp.int32)
    pk = (key & jnp.int32(~0xFFF)) | col
    big = jnp.int32(0x7FFFFFFF)
    tiles = [lax.slice(pk, (0, t * 128), (_BQ, (t + 1) * 128))
             for t in range(_N // 128)]
    regs = _top5_tree(tiles)
    guard = regs[_Q - 1]
    lane = lax.broadcasted_iota(jnp.int32, (_BQ, 128), 1)
    outs = []
    m = None
    for k in range(_NN + 1):
        m = jnp.min(regs[0], axis=1, keepdims=True)
        if k > 0:
            outs.append(m & jnp.int32(0xFFF))
        msk = lane == (m & jnp.int32(0x7F))
        for j in range(_Q - 1):
            regs[j] = jnp.where(msk, regs[j + 1], regs[j])
        regs[_Q - 1] = jnp.where(msk, big, regs[_Q - 1])
    idx_ref[0] = jnp.concatenate(outs, axis=1)
    unsafe = jnp.any(jnp.min(guard, axis=1, keepdims=True) < m)

    @pl.when(unsafe)
    def _slow():
        pks = pk
        souts = []
        for k in range(_NN + 1):
            sm = jnp.min(pks, axis=1, keepdims=True)
            if k > 0:
                souts.append(sm & jnp.int32(0xFFF))
            pks = jnp.where(pks == sm, big, pks)
        idx_ref[0] = jnp.concatenate(souts, axis=1)


def _knn_idx(points_ref):
    ptsp = jnp.pad(points_ref, ((0, 0), (0, 0), (0, _DP - _D)))
    ptsT = jnp.swapaxes(ptsp, 1, 2)
    return pl.pallas_call(
        _knn_body,
        grid=(_B, _N // _BQ),
        in_specs=[
            pl.BlockSpec((1, _BQ, _DP), lambda b, q: (b, q, 0)),
            pl.BlockSpec((1, _DP, _N), lambda b, q: (b, 0, 0)),
        ],
        out_specs=pl.BlockSpec((1, _BQ, _NN), lambda b, q: (b, q, 0)),
        out_shape=jax.ShapeDtypeStruct((_B, _N, _NN), jnp.int32),
    )(ptsp, ptsT)


def _rsqrt(x):
    i = lax.bitcast_convert_type(x, jnp.int32)
    y = lax.bitcast_convert_type(
        jnp.int32(0x5F3759DF) - lax.shift_right_logical(i, jnp.int32(1)),
        jnp.float32)
    for _ in range(3):
        y = y * (1.5 - 0.5 * x * y * y)
    return y


def _sc_loss_body(tpr_hbm, tp_hbm, idx_hbm, out_hbm,
                  pr_v, p_v, idx_v, acc_v):
    wid = lax.axis_index("s") * 2 + lax.axis_index("c")
    b = wid // _WPB
    e = wid % _WPB
    pltpu.sync_copy(tpr_hbm.at[pl.ds(b * _N * _DG, _N * _DG)], pr_v)
    pltpu.sync_copy(tp_hbm.at[pl.ds(b * _N * _DG, _N * _DG)], p_v)
    qbase = b * _N + e * _QW
    pltpu.sync_copy(idx_hbm.at[pl.ds(qbase * _NN, _QW * _NN)], idx_v)

    def row(r, acc):
        ridx = idx_v[pl.ds(r * _NN, _NN)] * _DG
        qoff = (e * _QW + r) * _DG
        s2r = jnp.zeros((16,), jnp.float32)
        s2 = jnp.zeros((16,), jnp.float32)
        for cd in range(_D):
            nbr_off = ridx + jnp.int32(cd)
            q_off = jnp.full((16,), qoff + cd, jnp.int32)
            dr = plsc.load_gather(pr_v, [nbr_off]) - \
                plsc.load_gather(pr_v, [q_off])
            s2r = s2r + dr * dr
            dp = plsc.load_gather(p_v, [nbr_off]) - \
                plsc.load_gather(p_v, [q_off])
            s2 = s2 + dp * dp
        dist_ref = s2r * _rsqrt(jnp.maximum(s2r, 1e-30))
        dist = s2 * _rsqrt(jnp.maximum(s2, 1e-30))
        stretch = jnp.maximum(dist / (dist_ref + 1e-10) - 1.0, 0.0)
        return acc + stretch

    acc = lax.fori_loop(0, _QW, row, jnp.zeros((16,), jnp.float32))
    acc_v[...] = acc
    pltpu.sync_copy(acc_v, out_hbm.at[wid])


def _sc_loss(table_pr, table_p, idx_flat):
    mesh = plsc.VectorSubcoreMesh(core_axis_name="c", subcore_axis_name="s")
    kfn = pl.kernel(
        _sc_loss_body,
        out_type=jax.ShapeDtypeStruct((_NW, 16), jnp.float32),
        mesh=mesh,
        compiler_params=pltpu.CompilerParams(needs_layout_passes=False),
        scratch_types=[
            pltpu.VMEM((_N * _DG,), jnp.float32),
            pltpu.VMEM((_N * _DG,), jnp.float32),
            pltpu.VMEM((_QW * _NN,), jnp.int32),
            pltpu.VMEM((16,), jnp.float32),
        ],
    )
    return kfn(table_pr, table_p, idx_flat)


def kernel(points_ref, points):
    gidx = _knn_idx(points_ref)
    table_pr = jnp.pad(points_ref.reshape(_B * _N, _D),
                       ((0, 0), (0, _DG - _D))).reshape(-1)
    table_p = jnp.pad(points.reshape(_B * _N, _D),
                      ((0, 0), (0, _DG - _D))).reshape(-1)
    partials = _sc_loss(table_pr, table_p, gidx.reshape(-1))
    return jnp.sum(partials) / jnp.float32(_B * _N * _NN)
